# baseline (speedup 1.0000x reference)
import jax
import jax.numpy as jnp
from jax import lax
from jax.experimental import pallas as pl
from jax.experimental.pallas import tpu as pltpu
from jax.experimental.pallas import tpu_sc as plsc

E = 8
K = 2
D = 768
M = 1024
T = 2048
BLK = 256
NB = T * K // BLK + E
R = NB * BLK

NC = 2
NS = 16
NW = NC * NS
TPW = T // NW


def _router_body(x_ref, wgt_ref, tok_ref, w0b_ref, w1b_ref, be_ref, xb16_ref):
    x = x_ref[...]
    wgt = wgt_ref[...]
    lg = lax.dot_general(wgt, x, (((0,), (1,)), ((), ())),
                         preferred_element_type=jnp.float32)
    eidx = lax.broadcasted_iota(jnp.int32, (E, T), 0)
    m1 = jnp.max(lg, axis=0, keepdims=True)
    i1 = jnp.min(jnp.where(lg == m1, eidx, E), axis=0, keepdims=True)
    oh1 = eidx == i1
    lg2 = jnp.where(oh1, jnp.float32(-jnp.inf), lg)
    m2 = jnp.max(lg2, axis=0, keepdims=True)
    i2 = jnp.min(jnp.where(lg2 == m2, eidx, E), axis=0, keepdims=True)
    oh2 = eidx == i2
    w1 = 1.0 / (1.0 + jnp.exp(m2 - m1))
    w2 = 1.0 - w1
    oh1f = oh1.astype(jnp.float32)
    oh2f = oh2.astype(jnp.float32)
    a = oh1f + oh2f
    CB = 256
    excl_chunks = []
    for c in range(T // CB):
        r_i = lax.broadcasted_iota(jnp.int32, (T, CB), 0)
        c_i = lax.broadcasted_iota(jnp.int32, (T, CB), 1) + c * CB
        tri = (r_i < c_i).astype(jnp.float32)
        excl_chunks.append(
            lax.dot_general(a, tri, (((1,), (0,)), ((), ())),
                            preferred_element_type=jnp.float32))
    excl = jnp.concatenate(excl_chunks, axis=1)
    counts = jnp.sum(a, axis=1, keepdims=True)
    pc = jnp.ceil(counts * (1.0 / BLK)) * BLK
    low = (lax.broadcasted_iota(jnp.int32, (E, E), 1)
           < lax.broadcasted_iota(jnp.int32, (E, E), 0)).astype(jnp.float32)
    pad_off = lax.dot_general(low, pc, (((1,), (0,)), ((), ())),
                              preferred_element_type=jnp.float32)
    rank0 = jnp.sum(oh1f * excl, axis=0, keepdims=True)
    rank1 = jnp.sum(oh2f * excl, axis=0, keepdims=True)
    off0 = jnp.sum(oh1f * pad_off, axis=0, keepdims=True)
    off1 = jnp.sum(oh2f * pad_off, axis=0, keepdims=True)
    row0 = (off0 + rank0).astype(jnp.int32)
    row1 = (off1 + rank1).astype(jnp.int32)
    tok_ref[...] = jnp.concatenate([row0, row1] * 4, axis=0)
    ones16 = jnp.ones((1, 16), jnp.float32)
    w0b_ref[...] = lax.dot_general(w1, ones16, (((0,), (0,)), ((), ())),
                                   preferred_element_type=jnp.float32)
    w1b_ref[...] = lax.dot_general(w2, ones16, (((0,), (0,)), ((), ())),
                                   preferred_element_type=jnp.float32)
    offb = pad_off * (1.0 / BLK)
    nbi = lax.broadcasted_iota(jnp.int32, (E, NB), 1).astype(jnp.float32)
    bes = jnp.sum((nbi >= offb).astype(jnp.float32), axis=0, keepdims=True) - 1.0
    be_ref[...] = jnp.concatenate([bes] * 8, axis=0).astype(jnp.int32)
    bits = lax.bitcast_convert_type(x, jnp.int32)
    rnd = bits + 0x7FFF + lax.shift_right_logical(bits, 16) % 2
    hi = lax.shift_right_logical(rnd, 16)
    xb16_ref[...] = (hi[:, : D // 2] & 0xFFFF) | (hi[:, D // 2:] << 16)


def _router(x, wgt):
    return pl.pallas_call(
        _router_body,
        out_shape=(
            jax.ShapeDtypeStruct((8, T), jnp.int32),
            jax.ShapeDtypeStruct((T, 16), jnp.float32),
            jax.ShapeDtypeStruct((T, 16), jnp.float32),
            jax.ShapeDtypeStruct((8, NB), jnp.int32),
            jax.ShapeDtypeStruct((T, D // 2), jnp.int32),
        ),
    )(x, wgt)


def _dispatch_body(x_hbm, tok_hbm, xs_hbm, xv, i0v, i1v, sem0, sem1):
    wid = lax.axis_index("s") * NC + lax.axis_index("c")
    base = wid * TPW
    pltpu.sync_copy(x_hbm.at[pl.ds(base, TPW)], xv)
    pltpu.sync_copy(tok_hbm.at[0, pl.ds(base, TPW)], i0v)
    pltpu.sync_copy(tok_hbm.at[1, pl.ds(base, TPW)], i1v)
    c0 = pltpu.async_copy(xv, xs_hbm.at[i0v], sem0)
    c1 = pltpu.async_copy(xv, xs_hbm.at[i1v], sem1)
    c0.wait()
    c1.wait()


def _dispatch(x, tok):
    return pl.kernel(
        _dispatch_body,
        out_type=jax.ShapeDtypeStruct((R, D // 2), jnp.int32),
        mesh=plsc.VectorSubcoreMesh(core_axis_name="c", subcore_axis_name="s",
                                    num_cores=NC, num_subcores=NS),
        scratch_types=[
            pltpu.VMEM((TPW, D // 2), jnp.int32),
            pltpu.VMEM((TPW,), jnp.int32),
            pltpu.VMEM((TPW,), jnp.int32),
            pltpu.SemaphoreType.DMA,
            pltpu.SemaphoreType.DMA,
        ],
    )(x, tok)


MB = 256
NMB = M // MB


def _mlp_body(be_ref, xs_ref, wi0_ref, wi1_ref, wo_ref, out_ref):
    j = pl.program_id(1)
    p = xs_ref[...]
    xlo = lax.bitcast_convert_type(p << 16, jnp.float32)
    xhi = lax.bitcast_convert_type(p & jnp.int32(-65536), jnp.float32)
    xb = jnp.concatenate([xlo, xhi], axis=1)
    h0 = jnp.dot(xb, wi0_ref[0], preferred_element_type=jnp.float32)
    h1 = jnp.dot(xb, wi1_ref[0], preferred_element_type=jnp.float32)
    act = h0 / (1.0 + jnp.exp(-h0)) * h1
    part = jnp.dot(act, wo_ref[0], preferred_element_type=jnp.float32)

    @pl.when(j == 0)
    def _init():
        out_ref[...] = part

    @pl.when(j != 0)
    def _acc():
        out_ref[...] += part


def _mlp(be, xs, wi_0, wi_1, wo):
    grid_spec = pltpu.PrefetchScalarGridSpec(
        num_scalar_prefetch=1,
        grid=(NB, NMB),
        in_specs=[
            pl.BlockSpec((BLK, D // 2), lambda i, j, be: (i, 0)),
            pl.BlockSpec((1, D, MB), lambda i, j, be: (be[0, i], 0, j)),
            pl.BlockSpec((1, D, MB), lambda i, j, be: (be[0, i], 0, j)),
            pl.BlockSpec((1, MB, D), lambda i, j, be: (be[0, i], j, 0)),
        ],
        out_specs=pl.BlockSpec((BLK, D), lambda i, j, be: (i, 0)),
    )
    return pl.pallas_call(
        _mlp_body,
        grid_spec=grid_spec,
        out_shape=jax.ShapeDtypeStruct((R, D), jnp.float32),
    )(be, xs, wi_0, wi_1, wo)


def _combine_body(ys_hbm, tok_hbm, w0_hbm, w1_hbm, out_hbm,
                  i0v, i1v, w0v, w1v, g0v, g1v, sem0, sem1):
    wid = lax.axis_index("s") * NC + lax.axis_index("c")
    base = wid * TPW
    pltpu.sync_copy(tok_hbm.at[0, pl.ds(base, TPW)], i0v)
    pltpu.sync_copy(tok_hbm.at[1, pl.ds(base, TPW)], i1v)
    pltpu.sync_copy(w0_hbm.at[pl.ds(base, TPW)], w0v)
    pltpu.sync_copy(w1_hbm.at[pl.ds(base, TPW)], w1v)
    g0 = pltpu.async_copy(ys_hbm.at[i0v], g0v, sem0)
    g1 = pltpu.async_copy(ys_hbm.at[i1v], g1v, sem1)
    g0.wait()
    g1.wait()

    def tbody(t, carry):
        wa = w0v[t, pl.ds(0, 16)]
        wb = w1v[t, pl.ds(0, 16)]
        for j in range(D // 16):
            sl = pl.ds(j * 16, 16)
            g0v[t, sl] = wa * g0v[t, sl] + wb * g1v[t, sl]
        return carry

    lax.fori_loop(0, TPW, tbody, 0)
    pltpu.sync_copy(g0v, out_hbm.at[pl.ds(base, TPW)])


def _combine(ys, tok, w0, w1):
    return pl.kernel(
        _combine_body,
        out_type=jax.ShapeDtypeStruct((T, D), jnp.float32),
        mesh=plsc.VectorSubcoreMesh(core_axis_name="c", subcore_axis_name="s",
                                    num_cores=NC, num_subcores=NS),
        scratch_types=[
            pltpu.VMEM((TPW,), jnp.int32),
            pltpu.VMEM((TPW,), jnp.int32),
            pltpu.VMEM((TPW, 16), jnp.float32),
            pltpu.VMEM((TPW, 16), jnp.float32),
            pltpu.VMEM((TPW, D), jnp.float32),
            pltpu.VMEM((TPW, D), jnp.float32),
            pltpu.SemaphoreType.DMA,
            pltpu.SemaphoreType.DMA,
        ],
    )(ys, tok, w0, w1)


def kernel(x, w_gate, wi_0, wi_1, wo):
    tok, w0b, w1b, bes, xpk = _router(x, w_gate)
    xs = _dispatch(xpk, tok)
    ys = _mlp(bes, xs, wi_0, wi_1, wo)
    return _combine(ys, tok, w0b, w1b)


# speedup vs baseline: 1.2234x; 1.2234x over previous
import jax
import jax.numpy as jnp
from jax import lax
from jax.experimental import pallas as pl
from jax.experimental.pallas import tpu as pltpu
from jax.experimental.pallas import tpu_sc as plsc

E = 8
K = 2
D = 768
M = 1024
T = 2048
BLK = 256
NB = T * K // BLK + E
R = NB * BLK

NC = 2
NS = 16
NW = NC * NS
TPW = T // NW


def _router_body(x_ref, wgt_ref, tok_ref, w0b_ref, w1b_ref, be_ref, xb16_ref):
    x = x_ref[...]
    wgt = wgt_ref[...]
    lg = lax.dot_general(wgt, x, (((0,), (1,)), ((), ())),
                         preferred_element_type=jnp.float32)
    eidx = lax.broadcasted_iota(jnp.int32, (E, T), 0)
    m1 = jnp.max(lg, axis=0, keepdims=True)
    i1 = jnp.min(jnp.where(lg == m1, eidx, E), axis=0, keepdims=True)
    oh1 = eidx == i1
    lg2 = jnp.where(oh1, jnp.float32(-jnp.inf), lg)
    m2 = jnp.max(lg2, axis=0, keepdims=True)
    i2 = jnp.min(jnp.where(lg2 == m2, eidx, E), axis=0, keepdims=True)
    oh2 = eidx == i2
    w1 = 1.0 / (1.0 + jnp.exp(m2 - m1))
    w2 = 1.0 - w1
    oh1f = oh1.astype(jnp.float32)
    oh2f = oh2.astype(jnp.float32)
    a = oh1f + oh2f
    CB = 256
    excl_chunks = []
    for c in range(T // CB):
        r_i = lax.broadcasted_iota(jnp.int32, (T, CB), 0)
        c_i = lax.broadcasted_iota(jnp.int32, (T, CB), 1) + c * CB
        tri = (r_i < c_i).astype(jnp.float32)
        excl_chunks.append(
            lax.dot_general(a, tri, (((1,), (0,)), ((), ())),
                            preferred_element_type=jnp.float32))
    excl = jnp.concatenate(excl_chunks, axis=1)
    counts = jnp.sum(a, axis=1, keepdims=True)
    pc = jnp.ceil(counts * (1.0 / BLK)) * BLK
    low = (lax.broadcasted_iota(jnp.int32, (E, E), 1)
           < lax.broadcasted_iota(jnp.int32, (E, E), 0)).astype(jnp.float32)
    pad_off = lax.dot_general(low, pc, (((1,), (0,)), ((), ())),
                              preferred_element_type=jnp.float32)
    rank0 = jnp.sum(oh1f * excl, axis=0, keepdims=True)
    rank1 = jnp.sum(oh2f * excl, axis=0, keepdims=True)
    off0 = jnp.sum(oh1f * pad_off, axis=0, keepdims=True)
    off1 = jnp.sum(oh2f * pad_off, axis=0, keepdims=True)
    row0 = (off0 + rank0).astype(jnp.int32)
    row1 = (off1 + rank1).astype(jnp.int32)
    tok_ref[...] = jnp.concatenate([row0, row1] * 4, axis=0)
    ones16 = jnp.ones((1, 16), jnp.float32)
    w0b_ref[...] = lax.dot_general(w1, ones16, (((0,), (0,)), ((), ())),
                                   preferred_element_type=jnp.float32)
    w1b_ref[...] = lax.dot_general(w2, ones16, (((0,), (0,)), ((), ())),
                                   preferred_element_type=jnp.float32)
    offb = pad_off * (1.0 / BLK)
    nbi = lax.broadcasted_iota(jnp.int32, (E, NB), 1).astype(jnp.float32)
    bes = jnp.sum((nbi >= offb).astype(jnp.float32), axis=0, keepdims=True) - 1.0
    be_ref[...] = jnp.concatenate([bes] * 8, axis=0).astype(jnp.int32)
    bits = lax.bitcast_convert_type(x, jnp.int32)
    rnd = bits + 0x7FFF + lax.shift_right_logical(bits, 16) % 2
    hi = lax.shift_right_logical(rnd, 16)
    xb16_ref[...] = (hi[:, : D // 2] & 0xFFFF) | (hi[:, D // 2:] << 16)


def _router(x, wgt):
    return pl.pallas_call(
        _router_body,
        out_shape=(
            jax.ShapeDtypeStruct((8, T), jnp.int32),
            jax.ShapeDtypeStruct((T, 16), jnp.float32),
            jax.ShapeDtypeStruct((T, 16), jnp.float32),
            jax.ShapeDtypeStruct((8, NB), jnp.int32),
            jax.ShapeDtypeStruct((T, D // 2), jnp.int32),
        ),
    )(x, wgt)


def _dispatch_body(x_hbm, tok_hbm, xs_hbm, xv, i0v, i1v, sem0, sem1):
    wid = lax.axis_index("s") * NC + lax.axis_index("c")
    base = wid * TPW
    pltpu.sync_copy(x_hbm.at[pl.ds(base, TPW)], xv)
    pltpu.sync_copy(tok_hbm.at[0, pl.ds(base, TPW)], i0v)
    pltpu.sync_copy(tok_hbm.at[1, pl.ds(base, TPW)], i1v)
    c0 = pltpu.async_copy(xv, xs_hbm.at[i0v], sem0)
    c1 = pltpu.async_copy(xv, xs_hbm.at[i1v], sem1)
    c0.wait()
    c1.wait()


def _dispatch(x, tok):
    return pl.kernel(
        _dispatch_body,
        out_type=jax.ShapeDtypeStruct((R, D // 2), jnp.int32),
        mesh=plsc.VectorSubcoreMesh(core_axis_name="c", subcore_axis_name="s",
                                    num_cores=NC, num_subcores=NS),
        scratch_types=[
            pltpu.VMEM((TPW, D // 2), jnp.int32),
            pltpu.VMEM((TPW,), jnp.int32),
            pltpu.VMEM((TPW,), jnp.int32),
            pltpu.SemaphoreType.DMA,
            pltpu.SemaphoreType.DMA,
        ],
    )(x, tok)


MB = 512
NMB = M // MB


def _mlp_body(be_ref, xs_ref, wi0_ref, wi1_ref, wo_ref, out_ref, xb_ref):
    j = pl.program_id(1)

    @pl.when(j == 0)
    def _unpack():
        p = xs_ref[...]
        xlo = lax.bitcast_convert_type(p << 16, jnp.float32)
        xhi = lax.bitcast_convert_type(p & jnp.int32(-65536), jnp.float32)
        xb_ref[...] = jnp.concatenate([xlo, xhi], axis=1)

    xb = xb_ref[...]
    h0 = jnp.dot(xb, wi0_ref[0], preferred_element_type=jnp.float32)
    h1 = jnp.dot(xb, wi1_ref[0], preferred_element_type=jnp.float32)
    act = h0 / (1.0 + jnp.exp(-h0)) * h1
    part = jnp.dot(act, wo_ref[0], preferred_element_type=jnp.float32)

    @pl.when(j == 0)
    def _init():
        out_ref[...] = part

    @pl.when(j != 0)
    def _acc():
        out_ref[...] += part


def _mlp(be, xs, wi_0, wi_1, wo):
    grid_spec = pltpu.PrefetchScalarGridSpec(
        num_scalar_prefetch=1,
        grid=(NB, NMB),
        in_specs=[
            pl.BlockSpec((BLK, D // 2), lambda i, j, be: (i, 0)),
            pl.BlockSpec((1, D, MB), lambda i, j, be: (be[0, i], 0, j)),
            pl.BlockSpec((1, D, MB), lambda i, j, be: (be[0, i], 0, j)),
            pl.BlockSpec((1, MB, D), lambda i, j, be: (be[0, i], j, 0)),
        ],
        out_specs=pl.BlockSpec((BLK, D), lambda i, j, be: (i, 0)),
        scratch_shapes=[pltpu.VMEM((BLK, D), jnp.float32)],
    )
    return pl.pallas_call(
        _mlp_body,
        grid_spec=grid_spec,
        out_shape=jax.ShapeDtypeStruct((R, D), jnp.float32),
    )(be, xs, wi_0, wi_1, wo)


def _combine_body(ys_hbm, tok_hbm, w0_hbm, w1_hbm, out_hbm,
                  i0v, i1v, w0v, w1v, g0v, g1v, sem0, sem1):
    wid = lax.axis_index("s") * NC + lax.axis_index("c")
    base = wid * TPW
    pltpu.sync_copy(tok_hbm.at[0, pl.ds(base, TPW)], i0v)
    pltpu.sync_copy(tok_hbm.at[1, pl.ds(base, TPW)], i1v)
    pltpu.sync_copy(w0_hbm.at[pl.ds(base, TPW)], w0v)
    pltpu.sync_copy(w1_hbm.at[pl.ds(base, TPW)], w1v)
    g0 = pltpu.async_copy(ys_hbm.at[i0v], g0v, sem0)
    g1 = pltpu.async_copy(ys_hbm.at[i1v], g1v, sem1)
    g0.wait()
    g1.wait()

    def tbody(t, carry):
        wa = w0v[t, pl.ds(0, 16)]
        wb = w1v[t, pl.ds(0, 16)]
        for j in range(D // 16):
            sl = pl.ds(j * 16, 16)
            g0v[t, sl] = wa * g0v[t, sl] + wb * g1v[t, sl]
        return carry

    lax.fori_loop(0, TPW, tbody, 0)
    pltpu.sync_copy(g0v, out_hbm.at[pl.ds(base, TPW)])


def _combine(ys, tok, w0, w1):
    return pl.kernel(
        _combine_body,
        out_type=jax.ShapeDtypeStruct((T, D), jnp.float32),
        mesh=plsc.VectorSubcoreMesh(core_axis_name="c", subcore_axis_name="s",
                                    num_cores=NC, num_subcores=NS),
        scratch_types=[
            pltpu.VMEM((TPW,), jnp.int32),
            pltpu.VMEM((TPW,), jnp.int32),
            pltpu.VMEM((TPW, 16), jnp.float32),
            pltpu.VMEM((TPW, 16), jnp.float32),
            pltpu.VMEM((TPW, D), jnp.float32),
            pltpu.VMEM((TPW, D), jnp.float32),
            pltpu.SemaphoreType.DMA,
            pltpu.SemaphoreType.DMA,
        ],
    )(ys, tok, w0, w1)


def kernel(x, w_gate, wi_0, wi_1, wo):
    tok, w0b, w1b, bes, xpk = _router(x, w_gate)
    xs = _dispatch(xpk, tok)
    ys = _mlp(bes, xs, wi_0, wi_1, wo)
    return _combine(ys, tok, w0b, w1b)


# speedup vs baseline: 1.6999x; 1.3894x over previous
import jax
import jax.numpy as jnp
from jax import lax
from jax.experimental import pallas as pl
from jax.experimental.pallas import tpu as pltpu
from jax.experimental.pallas import tpu_sc as plsc

E = 8
K = 2
D = 768
M = 1024
T = 2048
BLK = 256
NB = T * K // BLK + E
R = NB * BLK

NC = 2
NS = 16
NW = NC * NS
TPW = T // NW


def _router_body(x_ref, wgt_ref, tok_ref, w0b_ref, w1b_ref, be_ref, xb16_ref):
    x = x_ref[...]
    wgt = wgt_ref[...]
    lg = lax.dot_general(wgt, x, (((0,), (1,)), ((), ())),
                         preferred_element_type=jnp.float32)
    eidx = lax.broadcasted_iota(jnp.int32, (E, T), 0)
    m1 = jnp.max(lg, axis=0, keepdims=True)
    i1 = jnp.min(jnp.where(lg == m1, eidx, E), axis=0, keepdims=True)
    oh1 = eidx == i1
    lg2 = jnp.where(oh1, jnp.float32(-jnp.inf), lg)
    m2 = jnp.max(lg2, axis=0, keepdims=True)
    i2 = jnp.min(jnp.where(lg2 == m2, eidx, E), axis=0, keepdims=True)
    oh2 = eidx == i2
    w1 = 1.0 / (1.0 + jnp.exp(m2 - m1))
    w2 = 1.0 - w1
    oh1f = oh1.astype(jnp.float32)
    oh2f = oh2.astype(jnp.float32)
    a = oh1f + oh2f
    CB = 256
    excl_chunks = []
    for c in range(T // CB):
        r_i = lax.broadcasted_iota(jnp.int32, (T, CB), 0)
        c_i = lax.broadcasted_iota(jnp.int32, (T, CB), 1) + c * CB
        tri = (r_i < c_i).astype(jnp.float32)
        excl_chunks.append(
            lax.dot_general(a, tri, (((1,), (0,)), ((), ())),
                            preferred_element_type=jnp.float32))
    excl = jnp.concatenate(excl_chunks, axis=1)
    counts = jnp.sum(a, axis=1, keepdims=True)
    pc = jnp.maximum(jnp.ceil(counts * (1.0 / BLK)), 1.0) * BLK
    low = (lax.broadcasted_iota(jnp.int32, (E, E), 1)
           < lax.broadcasted_iota(jnp.int32, (E, E), 0)).astype(jnp.float32)
    pad_off = lax.dot_general(low, pc, (((1,), (0,)), ((), ())),
                              preferred_element_type=jnp.float32)
    rank0 = jnp.sum(oh1f * excl, axis=0, keepdims=True)
    rank1 = jnp.sum(oh2f * excl, axis=0, keepdims=True)
    off0 = jnp.sum(oh1f * pad_off, axis=0, keepdims=True)
    off1 = jnp.sum(oh2f * pad_off, axis=0, keepdims=True)
    row0 = (off0 + rank0).astype(jnp.int32)
    row1 = (off1 + rank1).astype(jnp.int32)
    tok_ref[...] = jnp.concatenate([row0, row1] * 4, axis=0)
    ones16 = jnp.ones((1, 16), jnp.float32)
    w0b_ref[...] = lax.dot_general(w1, ones16, (((0,), (0,)), ((), ())),
                                   preferred_element_type=jnp.float32)
    w1b_ref[...] = lax.dot_general(w2, ones16, (((0,), (0,)), ((), ())),
                                   preferred_element_type=jnp.float32)
    offb = pad_off * (1.0 / BLK)
    nbi = lax.broadcasted_iota(jnp.int32, (E, NB), 1).astype(jnp.float32)
    bes = jnp.sum((nbi >= offb).astype(jnp.float32), axis=0, keepdims=True) - 1.0
    fst = jnp.sum((nbi == offb).astype(jnp.float32), axis=0, keepdims=True)
    be_ref[...] = jnp.concatenate([bes, fst] * 4, axis=0).astype(jnp.int32)
    bits = lax.bitcast_convert_type(x, jnp.int32)
    rnd = bits + 0x7FFF + lax.shift_right_logical(bits, 16) % 2
    hi = lax.shift_right_logical(rnd, 16)
    xb16_ref[...] = (hi[:, : D // 2] & 0xFFFF) | (hi[:, D // 2:] << 16)


def _router(x, wgt):
    return pl.pallas_call(
        _router_body,
        out_shape=(
            jax.ShapeDtypeStruct((8, T), jnp.int32),
            jax.ShapeDtypeStruct((T, 16), jnp.float32),
            jax.ShapeDtypeStruct((T, 16), jnp.float32),
            jax.ShapeDtypeStruct((8, NB), jnp.int32),
            jax.ShapeDtypeStruct((T, D // 2), jnp.int32),
        ),
    )(x, wgt)


def _dispatch_body(x_hbm, tok_hbm, xs_hbm, xv, i0v, i1v, sem0, sem1):
    wid = lax.axis_index("s") * NC + lax.axis_index("c")
    base = wid * TPW
    pltpu.sync_copy(x_hbm.at[pl.ds(base, TPW)], xv)
    pltpu.sync_copy(tok_hbm.at[0, pl.ds(base, TPW)], i0v)
    pltpu.sync_copy(tok_hbm.at[1, pl.ds(base, TPW)], i1v)
    c0 = pltpu.async_copy(xv, xs_hbm.at[i0v], sem0)
    c1 = pltpu.async_copy(xv, xs_hbm.at[i1v], sem1)
    c0.wait()
    c1.wait()


def _dispatch(x, tok):
    return pl.kernel(
        _dispatch_body,
        out_type=jax.ShapeDtypeStruct((R, D // 2), jnp.int32),
        mesh=plsc.VectorSubcoreMesh(core_axis_name="c", subcore_axis_name="s",
                                    num_cores=NC, num_subcores=NS),
        scratch_types=[
            pltpu.VMEM((TPW, D // 2), jnp.int32),
            pltpu.VMEM((TPW,), jnp.int32),
            pltpu.VMEM((TPW,), jnp.int32),
            pltpu.SemaphoreType.DMA,
            pltpu.SemaphoreType.DMA,
        ],
    )(x, tok)


def _mlp_body(be_ref, xs_ref, wi0_hbm, wi1_hbm, wo_hbm, out_ref,
              w0a, w1a, woa, w0b, w1b, wob, sema, semb):
    i = pl.program_id(0)
    e = be_ref[0, i]
    first = be_ref[1, i]

    def _start(eidx, b0, b1, bo, sem):
        pltpu.make_async_copy(wi0_hbm.at[eidx], b0, sem).start()
        pltpu.make_async_copy(wi1_hbm.at[eidx], b1, sem).start()
        pltpu.make_async_copy(wo_hbm.at[eidx], bo, sem).start()

    def _wait(b0, b1, bo, sem):
        pltpu.make_async_copy(wi0_hbm.at[0], b0, sem).wait()
        pltpu.make_async_copy(wi1_hbm.at[0], b1, sem).wait()
        pltpu.make_async_copy(wo_hbm.at[0], bo, sem).wait()

    @pl.when(i == 0)
    def _warm():
        _start(0, w0a, w1a, woa, sema)
        _start(1, w0b, w1b, wob, semb)

    ep = e + 1

    @pl.when((first == 1) & (i > 0) & (e < 7) & (lax.rem(ep, 2) == 0))
    def _prefa():
        _start(ep, w0a, w1a, woa, sema)

    @pl.when((first == 1) & (i > 0) & (e < 7) & (lax.rem(ep, 2) == 1))
    def _prefb():
        _start(ep, w0b, w1b, wob, semb)

    @pl.when((first == 1) & (lax.rem(e, 2) == 0))
    def _waita():
        _wait(w0a, w1a, woa, sema)

    @pl.when((first == 1) & (lax.rem(e, 2) == 1))
    def _waitb():
        _wait(w0b, w1b, wob, semb)

    p = xs_ref[...]
    xlo = lax.bitcast_convert_type(p << 16, jnp.float32)
    xhi = lax.bitcast_convert_type(p & jnp.int32(-65536), jnp.float32)
    xb = jnp.concatenate([xlo, xhi], axis=1)

    def _emit(b0, b1, bo):
        h0 = jnp.dot(xb, b0[...], preferred_element_type=jnp.float32)
        h1 = jnp.dot(xb, b1[...], preferred_element_type=jnp.float32)
        act = h0 / (1.0 + jnp.exp(-h0)) * h1
        out_ref[...] = jnp.dot(act, bo[...],
                               preferred_element_type=jnp.float32)

    @pl.when(lax.rem(e, 2) == 0)
    def _compa():
        _emit(w0a, w1a, woa)

    @pl.when(lax.rem(e, 2) == 1)
    def _compb():
        _emit(w0b, w1b, wob)


def _mlp(be, xs, wi_0, wi_1, wo):
    grid_spec = pltpu.PrefetchScalarGridSpec(
        num_scalar_prefetch=1,
        grid=(NB,),
        in_specs=[
            pl.BlockSpec((BLK, D // 2), lambda i, be: (i, 0)),
            pl.BlockSpec(memory_space=pl.ANY),
            pl.BlockSpec(memory_space=pl.ANY),
            pl.BlockSpec(memory_space=pl.ANY),
        ],
        out_specs=pl.BlockSpec((BLK, D), lambda i, be: (i, 0)),
        scratch_shapes=[
            pltpu.VMEM((D, M), jnp.float32),
            pltpu.VMEM((D, M), jnp.float32),
            pltpu.VMEM((M, D), jnp.float32),
            pltpu.VMEM((D, M), jnp.float32),
            pltpu.VMEM((D, M), jnp.float32),
            pltpu.VMEM((M, D), jnp.float32),
            pltpu.SemaphoreType.DMA,
            pltpu.SemaphoreType.DMA,
        ],
    )
    return pl.pallas_call(
        _mlp_body,
        grid_spec=grid_spec,
        out_shape=jax.ShapeDtypeStruct((R, D), jnp.float32),
    )(be, xs, wi_0, wi_1, wo)


def _combine_body(ys_hbm, tok_hbm, w0_hbm, w1_hbm, out_hbm,
                  i0v, i1v, w0v, w1v, g0v, g1v, sem0, sem1):
    wid = lax.axis_index("s") * NC + lax.axis_index("c")
    base = wid * TPW
    pltpu.sync_copy(tok_hbm.at[0, pl.ds(base, TPW)], i0v)
    pltpu.sync_copy(tok_hbm.at[1, pl.ds(base, TPW)], i1v)
    pltpu.sync_copy(w0_hbm.at[pl.ds(base, TPW)], w0v)
    pltpu.sync_copy(w1_hbm.at[pl.ds(base, TPW)], w1v)
    g0 = pltpu.async_copy(ys_hbm.at[i0v], g0v, sem0)
    g1 = pltpu.async_copy(ys_hbm.at[i1v], g1v, sem1)
    g0.wait()
    g1.wait()

    def tbody(t, carry):
        wa = w0v[t, pl.ds(0, 16)]
        wb = w1v[t, pl.ds(0, 16)]
        for j in range(D // 16):
            sl = pl.ds(j * 16, 16)
            g0v[t, sl] = wa * g0v[t, sl] + wb * g1v[t, sl]
        return carry

    lax.fori_loop(0, TPW, tbody, 0)
    pltpu.sync_copy(g0v, out_hbm.at[pl.ds(base, TPW)])


def _combine(ys, tok, w0, w1):
    return pl.kernel(
        _combine_body,
        out_type=jax.ShapeDtypeStruct((T, D), jnp.float32),
        mesh=plsc.VectorSubcoreMesh(core_axis_name="c", subcore_axis_name="s",
                                    num_cores=NC, num_subcores=NS),
        scratch_types=[
            pltpu.VMEM((TPW,), jnp.int32),
            pltpu.VMEM((TPW,), jnp.int32),
            pltpu.VMEM((TPW, 16), jnp.float32),
            pltpu.VMEM((TPW, 16), jnp.float32),
            pltpu.VMEM((TPW, D), jnp.float32),
            pltpu.VMEM((TPW, D), jnp.float32),
            pltpu.SemaphoreType.DMA,
            pltpu.SemaphoreType.DMA,
        ],
    )(ys, tok, w0, w1)


def kernel(x, w_gate, wi_0, wi_1, wo):
    tok, w0b, w1b, bes, xpk = _router(x, w_gate)
    xs = _dispatch(xpk, tok)
    ys = _mlp(bes, xs, wi_0, wi_1, wo)
    return _combine(ys, tok, w0b, w1b)


# speedup vs baseline: 1.7447x; 1.0264x over previous
import jax
import jax.numpy as jnp
from jax import lax
from jax.experimental import pallas as pl
from jax.experimental.pallas import tpu as pltpu
from jax.experimental.pallas import tpu_sc as plsc

E = 8
K = 2
D = 768
M = 1024
T = 2048
BLK = 256
NB = T * K // BLK + E
R = NB * BLK

NC = 2
NS = 16
NW = NC * NS
TPW = T // NW


def _router_body(x_ref, wgt_ref, tok_ref, w0b_ref, w1b_ref, be_ref, xb16_ref):
    x = x_ref[...]
    wgt = wgt_ref[...]
    lg = lax.dot_general(wgt, x, (((0,), (1,)), ((), ())),
                         preferred_element_type=jnp.float32)
    eidx = lax.broadcasted_iota(jnp.int32, (E, T), 0)
    m1 = jnp.max(lg, axis=0, keepdims=True)
    i1 = jnp.min(jnp.where(lg == m1, eidx, E), axis=0, keepdims=True)
    oh1 = eidx == i1
    lg2 = jnp.where(oh1, jnp.float32(-jnp.inf), lg)
    m2 = jnp.max(lg2, axis=0, keepdims=True)
    i2 = jnp.min(jnp.where(lg2 == m2, eidx, E), axis=0, keepdims=True)
    oh2 = eidx == i2
    w1 = 1.0 / (1.0 + jnp.exp(m2 - m1))
    w2 = 1.0 - w1
    oh1f = oh1.astype(jnp.float32)
    oh2f = oh2.astype(jnp.float32)
    a = oh1f + oh2f
    CB = 256
    excl_chunks = []
    for c in range(T // CB):
        r_i = lax.broadcasted_iota(jnp.int32, (T, CB), 0)
        c_i = lax.broadcasted_iota(jnp.int32, (T, CB), 1) + c * CB
        tri = (r_i < c_i).astype(jnp.float32)
        excl_chunks.append(
            lax.dot_general(a, tri, (((1,), (0,)), ((), ())),
                            preferred_element_type=jnp.float32))
    excl = jnp.concatenate(excl_chunks, axis=1)
    counts = jnp.sum(a, axis=1, keepdims=True)
    pc = jnp.maximum(jnp.ceil(counts * (1.0 / BLK)), 1.0) * BLK
    low = (lax.broadcasted_iota(jnp.int32, (E, E), 1)
           < lax.broadcasted_iota(jnp.int32, (E, E), 0)).astype(jnp.float32)
    pad_off = lax.dot_general(low, pc, (((1,), (0,)), ((), ())),
                              preferred_element_type=jnp.float32)
    rank0 = jnp.sum(oh1f * excl, axis=0, keepdims=True)
    rank1 = jnp.sum(oh2f * excl, axis=0, keepdims=True)
    off0 = jnp.sum(oh1f * pad_off, axis=0, keepdims=True)
    off1 = jnp.sum(oh2f * pad_off, axis=0, keepdims=True)
    row0 = (off0 + rank0).astype(jnp.int32)
    row1 = (off1 + rank1).astype(jnp.int32)
    tok_ref[...] = jnp.concatenate([row0, row1] * 4, axis=0)
    ones16 = jnp.ones((1, 16), jnp.float32)
    w0b_ref[...] = lax.dot_general(w1, ones16, (((0,), (0,)), ((), ())),
                                   preferred_element_type=jnp.float32)
    w1b_ref[...] = lax.dot_general(w2, ones16, (((0,), (0,)), ((), ())),
                                   preferred_element_type=jnp.float32)
    offb = pad_off * (1.0 / BLK)
    nbi = lax.broadcasted_iota(jnp.int32, (E, NB), 1).astype(jnp.float32)
    bes = jnp.sum((nbi >= offb).astype(jnp.float32), axis=0, keepdims=True) - 1.0
    fst = jnp.sum((nbi == offb).astype(jnp.float32), axis=0, keepdims=True)
    be_ref[...] = jnp.concatenate([bes, fst] * 4, axis=0).astype(jnp.int32)
    bits = lax.bitcast_convert_type(x, jnp.int32)
    hi = lax.shift_right_logical(bits, 16)
    xb16_ref[...] = (hi[:, : D // 2] & 0xFFFF) | (hi[:, D // 2:] << 16)


def _router(x, wgt):
    return pl.pallas_call(
        _router_body,
        out_shape=(
            jax.ShapeDtypeStruct((8, T), jnp.int32),
            jax.ShapeDtypeStruct((T, 16), jnp.float32),
            jax.ShapeDtypeStruct((T, 16), jnp.float32),
            jax.ShapeDtypeStruct((8, NB), jnp.int32),
            jax.ShapeDtypeStruct((T, D // 2), jnp.int32),
        ),
    )(x, wgt)


def _dispatch_body(x_hbm, tok_hbm, xs_hbm, xv, i0v, i1v, sem0, sem1):
    wid = lax.axis_index("s") * NC + lax.axis_index("c")
    base = wid * TPW
    pltpu.sync_copy(x_hbm.at[pl.ds(base, TPW)], xv)
    pltpu.sync_copy(tok_hbm.at[0, pl.ds(base, TPW)], i0v)
    pltpu.sync_copy(tok_hbm.at[1, pl.ds(base, TPW)], i1v)
    c0 = pltpu.async_copy(xv, xs_hbm.at[i0v], sem0)
    c1 = pltpu.async_copy(xv, xs_hbm.at[i1v], sem1)
    c0.wait()
    c1.wait()


def _dispatch(x, tok):
    return pl.kernel(
        _dispatch_body,
        out_type=jax.ShapeDtypeStruct((R, D // 2), jnp.int32),
        mesh=plsc.VectorSubcoreMesh(core_axis_name="c", subcore_axis_name="s",
                                    num_cores=NC, num_subcores=NS),
        scratch_types=[
            pltpu.VMEM((TPW, D // 2), jnp.int32),
            pltpu.VMEM((TPW,), jnp.int32),
            pltpu.VMEM((TPW,), jnp.int32),
            pltpu.SemaphoreType.DMA,
            pltpu.SemaphoreType.DMA,
        ],
    )(x, tok)


def _mlp_body(be_ref, xs_ref, wi0_hbm, wi1_hbm, wo_hbm, out_ref,
              w0a, w1a, woa, w0b, w1b, wob, sema, semb):
    i = pl.program_id(0)
    e = be_ref[0, i]
    first = be_ref[1, i]

    def _start(eidx, b0, b1, bo, sem):
        pltpu.make_async_copy(wi0_hbm.at[eidx], b0, sem).start()
        pltpu.make_async_copy(wi1_hbm.at[eidx], b1, sem).start()
        pltpu.make_async_copy(wo_hbm.at[eidx], bo, sem).start()

    def _wait(b0, b1, bo, sem):
        pltpu.make_async_copy(wi0_hbm.at[0], b0, sem).wait()
        pltpu.make_async_copy(wi1_hbm.at[0], b1, sem).wait()
        pltpu.make_async_copy(wo_hbm.at[0], bo, sem).wait()

    @pl.when(i == 0)
    def _warm():
        _start(0, w0a, w1a, woa, sema)
        _start(1, w0b, w1b, wob, semb)

    ep = e + 1

    @pl.when((first == 1) & (i > 0) & (e < 7) & (lax.rem(ep, 2) == 0))
    def _prefa():
        _start(ep, w0a, w1a, woa, sema)

    @pl.when((first == 1) & (i > 0) & (e < 7) & (lax.rem(ep, 2) == 1))
    def _prefb():
        _start(ep, w0b, w1b, wob, semb)

    @pl.when((first == 1) & (lax.rem(e, 2) == 0))
    def _waita():
        _wait(w0a, w1a, woa, sema)

    @pl.when((first == 1) & (lax.rem(e, 2) == 1))
    def _waitb():
        _wait(w0b, w1b, wob, semb)

    p = xs_ref[...]
    xlo = lax.bitcast_convert_type(p << 16, jnp.float32)
    xhi = lax.bitcast_convert_type(p & jnp.int32(-65536), jnp.float32)
    xb = jnp.concatenate([xlo, xhi], axis=1)

    def _emit(b0, b1, bo):
        h0 = jnp.dot(xb, b0[...], preferred_element_type=jnp.float32)
        h1 = jnp.dot(xb, b1[...], preferred_element_type=jnp.float32)
        act = h0 / (1.0 + jnp.exp(-h0)) * h1
        out_ref[...] = jnp.dot(act, bo[...],
                               preferred_element_type=jnp.float32)

    @pl.when(lax.rem(e, 2) == 0)
    def _compa():
        _emit(w0a, w1a, woa)

    @pl.when(lax.rem(e, 2) == 1)
    def _compb():
        _emit(w0b, w1b, wob)


def _mlp(be, xs, wi_0, wi_1, wo):
    grid_spec = pltpu.PrefetchScalarGridSpec(
        num_scalar_prefetch=1,
        grid=(NB,),
        in_specs=[
            pl.BlockSpec((BLK, D // 2), lambda i, be: (i, 0)),
            pl.BlockSpec(memory_space=pl.ANY),
            pl.BlockSpec(memory_space=pl.ANY),
            pl.BlockSpec(memory_space=pl.ANY),
        ],
        out_specs=pl.BlockSpec((BLK, D), lambda i, be: (i, 0)),
        scratch_shapes=[
            pltpu.VMEM((D, M), jnp.float32),
            pltpu.VMEM((D, M), jnp.float32),
            pltpu.VMEM((M, D), jnp.float32),
            pltpu.VMEM((D, M), jnp.float32),
            pltpu.VMEM((D, M), jnp.float32),
            pltpu.VMEM((M, D), jnp.float32),
            pltpu.SemaphoreType.DMA,
            pltpu.SemaphoreType.DMA,
        ],
    )
    return pl.pallas_call(
        _mlp_body,
        grid_spec=grid_spec,
        out_shape=jax.ShapeDtypeStruct((R, D), jnp.float32),
    )(be, xs, wi_0, wi_1, wo)


def _combine_body(ys_hbm, tok_hbm, w0_hbm, w1_hbm, out_hbm,
                  i0v, i1v, w0v, w1v, g0v, g1v, sem0, sem1):
    wid = lax.axis_index("s") * NC + lax.axis_index("c")
    base = wid * TPW
    pltpu.sync_copy(tok_hbm.at[0, pl.ds(base, TPW)], i0v)
    pltpu.sync_copy(tok_hbm.at[1, pl.ds(base, TPW)], i1v)
    pltpu.sync_copy(w0_hbm.at[pl.ds(base, TPW)], w0v)
    pltpu.sync_copy(w1_hbm.at[pl.ds(base, TPW)], w1v)
    g0 = pltpu.async_copy(ys_hbm.at[i0v], g0v, sem0)
    g1 = pltpu.async_copy(ys_hbm.at[i1v], g1v, sem1)
    g0.wait()
    g1.wait()

    @plsc.parallel_loop(0, TPW, step=1)
    def _tbody(t):
        wa = w0v[t, pl.ds(0, 16)]
        wb = w1v[t, pl.ds(0, 16)]
        for j in range(D // 16):
            sl = pl.ds(j * 16, 16)
            g0v[t, sl] = wa * g0v[t, sl] + wb * g1v[t, sl]

    pltpu.sync_copy(g0v, out_hbm.at[pl.ds(base, TPW)])


def _combine(ys, tok, w0, w1):
    return pl.kernel(
        _combine_body,
        out_type=jax.ShapeDtypeStruct((T, D), jnp.float32),
        mesh=plsc.VectorSubcoreMesh(core_axis_name="c", subcore_axis_name="s",
                                    num_cores=NC, num_subcores=NS),
        scratch_types=[
            pltpu.VMEM((TPW,), jnp.int32),
            pltpu.VMEM((TPW,), jnp.int32),
            pltpu.VMEM((TPW, 16), jnp.float32),
            pltpu.VMEM((TPW, 16), jnp.float32),
            pltpu.VMEM((TPW, D), jnp.float32),
            pltpu.VMEM((TPW, D), jnp.float32),
            pltpu.SemaphoreType.DMA,
            pltpu.SemaphoreType.DMA,
        ],
    )(ys, tok, w0, w1)


def kernel(x, w_gate, wi_0, wi_1, wo):
    tok, w0b, w1b, bes, xpk = _router(x, w_gate)
    xs = _dispatch(xpk, tok)
    ys = _mlp(bes, xs, wi_0, wi_1, wo)
    return _combine(ys, tok, w0b, w1b)
